# baseline (speedup 1.0000x reference)
import functools

import jax
import jax.numpy as jnp
from jax import lax
from jax.experimental import pallas as pl
from jax.experimental.pallas import tpu as pltpu
from jax.experimental.pallas import tpu_sc as plsc

B, N, D = 32, 576, 768
N_SLOTS = 8
LANES = 16
NV = N // LANES
NP = 640
RC = 5
CW = NP // RC


def _gram_body(f_ref, g_ref, sal_ref):
    f = f_ref[0]
    norm = jnp.sqrt(jnp.sum(f * f, axis=1, keepdims=True))
    fn = f / (norm + 1e-12)
    g = lax.dot_general(fn, fn, (((1,), (1,)), ((), ())),
                        preferred_element_type=jnp.float32)
    g_ref[0, :, :N] = g
    sal_ref[0, 0] = norm[:, 0]


def _tc_gram(features, off, hb):
    return pl.pallas_call(
        _gram_body,
        grid=(hb,),
        in_specs=[pl.BlockSpec((1, N, D), lambda b: (b + off, 0, 0))],
        out_specs=[
            pl.BlockSpec((1, N, NP), lambda b: (b, 0, 0)),
            pl.BlockSpec((1, 1, N), lambda b: (b, 0, 0)),
        ],
        out_shape=[
            jax.ShapeDtypeStruct((hb, N, NP), jnp.float32),
            jax.ShapeDtypeStruct((hb, 1, N), jnp.float32),
        ],
    )(features)


def _lane_gather(v, idx):
    return v.at[idx].get(mode="promise_in_bounds")


_UNROLL = 4
assert NV % _UNROLL == 0


def _merge(av, ai, bv, bi):
    better = (bv > av) | ((bv == av) & (bi < ai))
    return jnp.where(better, bv, av), jnp.where(better, bi, ai)


def _sc_greedy(hb, off, sal0_hbm, g_hbm, f_hbm, out_hbm, sal_v, grow_v,
               idx_v, slots_v, sem):
    b = lax.axis_index("s") * 2 + lax.axis_index("c")

    @pl.when(b < hb)
    def _():
        _sc_greedy_body(b, off, sal0_hbm, g_hbm, f_hbm, out_hbm, sal_v,
                        grow_v, idx_v, slots_v, sem)


def _sc_greedy_body(b, off, sal0_hbm, g_hbm, f_hbm, out_hbm, sal_v, grow_v,
                    idx_v, slots_v, sem):
    pltpu.sync_copy(sal0_hbm.at[b, 0], sal_v)
    iota = lax.iota(jnp.int32, LANES)
    neginf = jnp.float32(-jnp.inf)
    sel_vec = jnp.full((LANES,), b * N, jnp.int32)
    zero_i = jnp.zeros((LANES,), jnp.int32)
    ninf_v = jnp.full((LANES,), neginf)

    def argmax_lanes(carry_in, update_with_row, prev_idx):
        def body(j, carry):
            accs = list(carry)
            for u in range(_UNROLL):
                jj = j * _UNROLL + u
                v = sal_v[pl.ds(jj * LANES, LANES)]
                gi = jj * LANES + iota
                if update_with_row:
                    sim = grow_v[0, pl.ds(jj * LANES, LANES)]
                    factor = 1.0 - jnp.clip(sim, 0.0, 1.0)
                    keep_inf = (gi == prev_idx) | (v == neginf)
                    v = jnp.where(keep_inf, neginf, v * factor)
                    sal_v[pl.ds(jj * LANES, LANES)] = v
                av, ai = accs[2 * u], accs[2 * u + 1]
                upd = v > av
                accs[2 * u] = jnp.where(upd, v, av)
                accs[2 * u + 1] = jnp.where(upd, gi, ai)
            return tuple(accs)

        carry = lax.fori_loop(0, NV // _UNROLL, body, carry_in)
        vmax, vidx = carry[0], carry[1]
        for u in range(1, _UNROLL):
            vmax, vidx = _merge(vmax, vidx, carry[2 * u], carry[2 * u + 1])
        for k in (1, 2, 4, 8):
            pv = _lane_gather(vmax, iota ^ k)
            pi = _lane_gather(vidx, iota ^ k)
            vmax, vidx = _merge(vmax, vidx, pv, pi)
        return vidx

    init = tuple(x for _ in range(_UNROLL) for x in (ninf_v, zero_i))
    idx_bcast = argmax_lanes(init, False, None)
    for t in range(N_SLOTS):
        gidx_vec = idx_bcast + b * N
        sel_vec = jnp.where(iota == t, gidx_vec, sel_vec)
        if t == N_SLOTS - 1:
            break
        idx_v[...] = gidx_vec
        pltpu.async_copy(g_hbm.at[idx_v.at[pl.ds(0, 1)]], grow_v, sem).wait()
        idx_bcast = argmax_lanes(init, True, idx_bcast)

    idx_v[...] = sel_vec + off * N
    pltpu.async_copy(f_hbm.at[idx_v.at[pl.ds(0, N_SLOTS)]], slots_v,
                     sem).wait()
    pltpu.sync_copy(slots_v, out_hbm.at[b])


@functools.lru_cache(maxsize=4)
def _sc_greedy_kernel(hb, off):
    mesh = plsc.VectorSubcoreMesh(core_axis_name="c", subcore_axis_name="s",
                                  num_cores=2, num_subcores=16)
    return pl.kernel(
        functools.partial(_sc_greedy, hb, off),
        out_type=jax.ShapeDtypeStruct((hb, N_SLOTS, D), jnp.float32),
        mesh=mesh,
        scratch_types=[
            pltpu.VMEM((N,), jnp.float32),
            pltpu.VMEM((1, NP), jnp.float32),
            pltpu.VMEM((LANES,), jnp.int32),
            pltpu.VMEM((N_SLOTS, D), jnp.float32),
            pltpu.SemaphoreType.DMA,
        ],
    )


CHUNKS = 2
HB = B // CHUNKS


@jax.jit
def kernel(features):
    f2 = features.reshape(B * N, D)
    outs = []
    for c in range(CHUNKS):
        g, sal0 = _tc_gram(features, c * HB, HB)
        g2 = g.reshape(HB * N, NP)
        outs.append(_sc_greedy_kernel(HB, c * HB)(sal0, g2, f2))
    return jnp.concatenate(outs, axis=0)


# speedup vs baseline: 1.0052x; 1.0052x over previous
import functools

import jax
import jax.numpy as jnp
from jax import lax
from jax.experimental import pallas as pl
from jax.experimental.pallas import tpu as pltpu
from jax.experimental.pallas import tpu_sc as plsc

B, N, D = 32, 576, 768
N_SLOTS = 8
LANES = 16
NV = N // LANES
NP = 640
RC = 5
CW = NP // RC


def _gram_body(f_ref, g_ref, sal_ref):
    f = f_ref[0]
    norm = jnp.sqrt(jnp.sum(f * f, axis=1, keepdims=True))
    fn = f / (norm + 1e-12)
    g = lax.dot_general(fn, fn, (((1,), (1,)), ((), ())),
                        preferred_element_type=jnp.float32)
    g_ref[0, :, :N] = g
    sal_ref[0, 0] = norm[:, 0]


def _tc_gram(features, off, hb):
    return pl.pallas_call(
        _gram_body,
        grid=(hb,),
        in_specs=[pl.BlockSpec((1, N, D), lambda b: (b + off, 0, 0))],
        out_specs=[
            pl.BlockSpec((1, N, NP), lambda b: (b, 0, 0)),
            pl.BlockSpec((1, 1, N), lambda b: (b, 0, 0)),
        ],
        out_shape=[
            jax.ShapeDtypeStruct((hb, N, NP), jnp.float32),
            jax.ShapeDtypeStruct((hb, 1, N), jnp.float32),
        ],
    )(features)


def _lane_gather(v, idx):
    return v.at[idx].get(mode="promise_in_bounds")


_UNROLL = 4
assert NV % _UNROLL == 0


def _merge(av, ai, bv, bi):
    better = (bv > av) | ((bv == av) & (bi < ai))
    return jnp.where(better, bv, av), jnp.where(better, bi, ai)


def _sc_greedy(hb, off, sal0_hbm, g_hbm, f_hbm, out_hbm, sal_v, grow_v,
               idx_v, slots_v, sem):
    b = lax.axis_index("s") * 2 + lax.axis_index("c")

    @pl.when(b < hb)
    def _():
        _sc_greedy_body(b, off, sal0_hbm, g_hbm, f_hbm, out_hbm, sal_v,
                        grow_v, idx_v, slots_v, sem)


def _sc_greedy_body(b, off, sal0_hbm, g_hbm, f_hbm, out_hbm, sal_v, grow_v,
                    idx_v, slots_v, sem):
    pltpu.sync_copy(sal0_hbm.at[b, 0], sal_v)
    iota = lax.iota(jnp.int32, LANES)
    neginf = jnp.float32(-jnp.inf)
    sel_vec = jnp.full((LANES,), b * N, jnp.int32)
    zero_i = jnp.zeros((LANES,), jnp.int32)
    ninf_v = jnp.full((LANES,), neginf)

    def argmax_lanes(carry_in, update_with_row, prev_idx):
        def body(j, carry):
            accs = list(carry)
            for u in range(_UNROLL):
                jj = j * _UNROLL + u
                v = sal_v[pl.ds(jj * LANES, LANES)]
                gi = jj * LANES + iota
                if update_with_row:
                    sim = grow_v[0, pl.ds(jj * LANES, LANES)]
                    factor = 1.0 - jnp.clip(sim, 0.0, 1.0)
                    keep_inf = (gi == prev_idx) | (v == neginf)
                    v = jnp.where(keep_inf, neginf, v * factor)
                    sal_v[pl.ds(jj * LANES, LANES)] = v
                av, ai = accs[2 * u], accs[2 * u + 1]
                upd = v > av
                accs[2 * u] = jnp.where(upd, v, av)
                accs[2 * u + 1] = jnp.where(upd, gi, ai)
            return tuple(accs)

        carry = lax.fori_loop(0, NV // _UNROLL, body, carry_in)
        vmax, vidx = carry[0], carry[1]
        for u in range(1, _UNROLL):
            vmax, vidx = _merge(vmax, vidx, carry[2 * u], carry[2 * u + 1])
        for k in (1, 2, 4, 8):
            pv = _lane_gather(vmax, iota ^ k)
            pi = _lane_gather(vidx, iota ^ k)
            vmax, vidx = _merge(vmax, vidx, pv, pi)
        return vidx

    init = tuple(x for _ in range(_UNROLL) for x in (ninf_v, zero_i))
    idx_bcast = argmax_lanes(init, False, None)
    for t in range(N_SLOTS):
        gidx_vec = idx_bcast + b * N
        sel_vec = jnp.where(iota == t, gidx_vec, sel_vec)
        if t == N_SLOTS - 1:
            break
        idx_v[...] = gidx_vec
        pltpu.async_copy(g_hbm.at[idx_v.at[pl.ds(0, 1)]], grow_v, sem).wait()
        idx_bcast = argmax_lanes(init, True, idx_bcast)

    idx_v[...] = sel_vec + off * N
    pltpu.async_copy(f_hbm.at[idx_v.at[pl.ds(0, N_SLOTS)]], slots_v,
                     sem).wait()
    pltpu.sync_copy(slots_v, out_hbm.at[b])


@functools.lru_cache(maxsize=4)
def _sc_greedy_kernel(hb, off):
    mesh = plsc.VectorSubcoreMesh(core_axis_name="c", subcore_axis_name="s",
                                  num_cores=2, num_subcores=16)
    return pl.kernel(
        functools.partial(_sc_greedy, hb, off),
        out_type=jax.ShapeDtypeStruct((hb, N_SLOTS, D), jnp.float32),
        mesh=mesh,
        scratch_types=[
            pltpu.VMEM((N,), jnp.float32),
            pltpu.VMEM((1, NP), jnp.float32),
            pltpu.VMEM((LANES,), jnp.int32),
            pltpu.VMEM((N_SLOTS, D), jnp.float32),
            pltpu.SemaphoreType.DMA,
        ],
    )


CHUNKS = 2
HB = B // CHUNKS


@jax.jit
def kernel(features):
    f2 = features.reshape(B * N, D)
    grams = [_tc_gram(features, c * HB, HB) for c in range(CHUNKS)]
    outs = [
        _sc_greedy_kernel(HB, c * HB)(sal0, g.reshape(HB * N, NP), f2)
        for c, (g, sal0) in enumerate(grams)
    ]
    return jnp.concatenate(outs, axis=0)


# speedup vs baseline: 1.0644x; 1.0589x over previous
import functools

import jax
import jax.numpy as jnp
from jax import lax
from jax.experimental import pallas as pl
from jax.experimental.pallas import tpu as pltpu
from jax.experimental.pallas import tpu_sc as plsc

B, N, D = 32, 576, 768
N_SLOTS = 8
LANES = 16
NV = N // LANES
NP = 640
RC = 5
CW = NP // RC


def _gram_body(f_ref, g_ref, sal_ref):
    f = f_ref[0]
    norm = jnp.sqrt(jnp.sum(f * f, axis=1, keepdims=True))
    fn = f / (norm + 1e-12)
    g = lax.dot_general(fn, fn, (((1,), (1,)), ((), ())),
                        preferred_element_type=jnp.float32)
    g_ref[0, :, :N] = g
    sal_ref[0, 0] = norm[:, 0]


def _tc_gram(features, off, hb):
    return pl.pallas_call(
        _gram_body,
        grid=(hb,),
        in_specs=[pl.BlockSpec((1, N, D), lambda b: (b + off, 0, 0))],
        out_specs=[
            pl.BlockSpec((1, N, NP), lambda b: (b, 0, 0)),
            pl.BlockSpec((1, 1, N), lambda b: (b, 0, 0)),
        ],
        out_shape=[
            jax.ShapeDtypeStruct((hb, N, NP), jnp.float32),
            jax.ShapeDtypeStruct((hb, 1, N), jnp.float32),
        ],
    )(features)


def _lane_gather(v, idx):
    return v.at[idx].get(mode="promise_in_bounds")


_UNROLL = 4
assert NV % _UNROLL == 0


def _merge(av, ai, bv, bi):
    better = (bv > av) | ((bv == av) & (bi < ai))
    return jnp.where(better, bv, av), jnp.where(better, bi, ai)


def _sc_greedy(hb, off, sal0_hbm, g_hbm, f_hbm, out_hbm, sal_v, grow_v,
               idx_v, slots_v, sem):
    b = lax.axis_index("s") * 2 + lax.axis_index("c")

    @pl.when(b < hb)
    def _():
        _sc_greedy_body(b, off, sal0_hbm, g_hbm, f_hbm, out_hbm, sal_v,
                        grow_v, idx_v, slots_v, sem)


def _sc_greedy_body(b, off, sal0_hbm, g_hbm, f_hbm, out_hbm, sal_v, grow_v,
                    idx_v, slots_v, sem):
    pltpu.sync_copy(sal0_hbm.at[b, 0], sal_v)
    iota = lax.iota(jnp.int32, LANES)
    neginf = jnp.float32(-jnp.inf)
    sel_vec = jnp.full((LANES,), b * N, jnp.int32)
    zero_i = jnp.zeros((LANES,), jnp.int32)
    ninf_v = jnp.full((LANES,), neginf)

    def argmax_lanes(carry_in, update_with_row, prev_idx):
        def body(j, carry):
            accs = list(carry)
            for u in range(_UNROLL):
                jj = j * _UNROLL + u
                v = sal_v[pl.ds(jj * LANES, LANES)]
                gi = jj * LANES + iota
                if update_with_row:
                    sim = grow_v[0, pl.ds(jj * LANES, LANES)]
                    factor = 1.0 - jnp.clip(sim, 0.0, 1.0)
                    keep_inf = (gi == prev_idx) | (v == neginf)
                    v = jnp.where(keep_inf, neginf, v * factor)
                    sal_v[pl.ds(jj * LANES, LANES)] = v
                av, ai = accs[2 * u], accs[2 * u + 1]
                upd = v > av
                accs[2 * u] = jnp.where(upd, v, av)
                accs[2 * u + 1] = jnp.where(upd, gi, ai)
            return tuple(accs)

        carry = carry_in
        for j in range(NV // _UNROLL):
            carry = body(j, carry)
        vmax, vidx = carry[0], carry[1]
        for u in range(1, _UNROLL):
            vmax, vidx = _merge(vmax, vidx, carry[2 * u], carry[2 * u + 1])
        for k in (1, 2, 4, 8):
            pv = _lane_gather(vmax, iota ^ k)
            pi = _lane_gather(vidx, iota ^ k)
            vmax, vidx = _merge(vmax, vidx, pv, pi)
        return vidx

    init = tuple(x for _ in range(_UNROLL) for x in (ninf_v, zero_i))
    idx_bcast = argmax_lanes(init, False, None)
    for t in range(N_SLOTS):
        gidx_vec = idx_bcast + b * N
        sel_vec = jnp.where(iota == t, gidx_vec, sel_vec)
        if t == N_SLOTS - 1:
            break
        idx_v[...] = gidx_vec
        pltpu.async_copy(g_hbm.at[idx_v.at[pl.ds(0, 1)]], grow_v, sem).wait()
        idx_bcast = argmax_lanes(init, True, idx_bcast)

    idx_v[...] = sel_vec + off * N
    pltpu.async_copy(f_hbm.at[idx_v.at[pl.ds(0, N_SLOTS)]], slots_v,
                     sem).wait()
    pltpu.sync_copy(slots_v, out_hbm.at[b])


@functools.lru_cache(maxsize=4)
def _sc_greedy_kernel(hb, off):
    mesh = plsc.VectorSubcoreMesh(core_axis_name="c", subcore_axis_name="s",
                                  num_cores=2, num_subcores=16)
    return pl.kernel(
        functools.partial(_sc_greedy, hb, off),
        out_type=jax.ShapeDtypeStruct((hb, N_SLOTS, D), jnp.float32),
        mesh=mesh,
        scratch_types=[
            pltpu.VMEM((N,), jnp.float32),
            pltpu.VMEM((1, NP), jnp.float32),
            pltpu.VMEM((LANES,), jnp.int32),
            pltpu.VMEM((N_SLOTS, D), jnp.float32),
            pltpu.SemaphoreType.DMA,
        ],
    )


@jax.jit
def kernel(features):
    f2 = features.reshape(B * N, D)
    g, sal0 = _tc_gram(features, 0, B)
    return _sc_greedy_kernel(B, 0)(sal0, g.reshape(B * N, NP), f2)
